# Initial kernel scaffold; baseline (speedup 1.0000x reference)
import jax
import jax.numpy as jnp
from jax.experimental import pallas as pl


def kernel(x, edge_index, W0, b0, W1, b1, W2, b2, Wm0, bm0, Wm1, bm1):
    raise NotImplementedError("write your pallas kernel here")


# speedup vs baseline: 4.5311x; 4.5311x over previous
import functools

import jax
import jax.numpy as jnp
from jax import lax
from jax.experimental import pallas as pl
from jax.experimental.pallas import tpu as pltpu
from jax.experimental.pallas import tpu_sc as plsc

N = 10000
E = 320000
D = 128
MLP_H = 64
D_OUT = 40

NC = 2
NS = 16
NW = NC * NS
E_PER_W = E // NW
CHUNK = 80
NCHUNKS = E_PER_W // CHUNK
ROWS_A = 624
ROWS_LAST = N - (NS - 1) * ROWS_A

_mesh = plsc.VectorSubcoreMesh(
    core_axis_name="c", subcore_axis_name="s", num_cores=NC, num_subcores=NS
)


@functools.partial(
    pl.kernel,
    out_type=jax.ShapeDtypeStruct((NC * N, D), jnp.float32),
    mesh=_mesh,
    scratch_types=[
        pltpu.VMEM((CHUNK,), jnp.int32),
        pltpu.VMEM((CHUNK,), jnp.int32),
        pltpu.VMEM((CHUNK, D), jnp.float32),
        pltpu.VMEM_SHARED((N, D), jnp.float32),
        pltpu.SemaphoreType.DMA,
    ],
)
def _sc_scatter(h, src, dst, zeros_tile, out, sidx, didx, rows, acc, sem):
    cid = lax.axis_index("c")
    sid = lax.axis_index("s")
    wid = sid * NC + cid

    @pl.when(sid < NS - 1)
    def _():
        pltpu.sync_copy(zeros_tile.at[pl.ds(0, ROWS_A)],
                        acc.at[pl.ds(sid * ROWS_A, ROWS_A)])

    @pl.when(sid == NS - 1)
    def _():
        pltpu.sync_copy(zeros_tile, acc.at[pl.ds((NS - 1) * ROWS_A, ROWS_LAST)])

    plsc.subcore_barrier()

    ebase = wid * E_PER_W

    def body(j, carry):
        base = ebase + j * CHUNK
        pltpu.sync_copy(src.at[pl.ds(base, CHUNK)], sidx)
        pltpu.sync_copy(dst.at[pl.ds(base, CHUNK)], didx)
        pltpu.async_copy(h.at[sidx], rows, sem).wait()
        pltpu.sync_copy(rows, acc.at[didx], add=True)
        return carry

    lax.fori_loop(0, NCHUNKS, body, 0)
    plsc.subcore_barrier()

    @pl.when(sid < NS - 1)
    def _():
        pltpu.sync_copy(
            acc.at[pl.ds(sid * ROWS_A, ROWS_A)],
            out.at[pl.ds(cid * N + sid * ROWS_A, ROWS_A)],
        )

    @pl.when(sid == NS - 1)
    def _():
        pltpu.sync_copy(
            acc.at[pl.ds((NS - 1) * ROWS_A, ROWS_LAST)],
            out.at[pl.ds(cid * N + (NS - 1) * ROWS_A, ROWS_LAST)],
        )


BLK = 2000


def _mm_body(x_ref, w_ref, o_ref):
    o_ref[...] = jnp.dot(x_ref[...], w_ref[...], preferred_element_type=jnp.float32)


_mm = pl.pallas_call(
    _mm_body,
    grid=(N // BLK,),
    in_specs=[
        pl.BlockSpec((BLK, D), lambda i: (i, 0)),
        pl.BlockSpec((D, D), lambda i: (0, 0)),
    ],
    out_specs=pl.BlockSpec((BLK, D), lambda i: (i, 0)),
    out_shape=jax.ShapeDtypeStruct((N, D), jnp.float32),
)


def _agg_mm_body(p0_ref, p1_ref, b_ref, w_ref, o_ref):
    h = jnp.maximum(p0_ref[...] + p1_ref[...] + b_ref[...], 0.0)
    o_ref[...] = jnp.dot(h, w_ref[...], preferred_element_type=jnp.float32)


_agg_mm = pl.pallas_call(
    _agg_mm_body,
    grid=(N // BLK,),
    in_specs=[
        pl.BlockSpec((BLK, D), lambda i: (i, 0)),
        pl.BlockSpec((BLK, D), lambda i: (N // BLK + i, 0)),
        pl.BlockSpec((1, D), lambda i: (0, 0)),
        pl.BlockSpec((D, D), lambda i: (0, 0)),
    ],
    out_specs=pl.BlockSpec((BLK, D), lambda i: (i, 0)),
    out_shape=jax.ShapeDtypeStruct((N, D), jnp.float32),
)


def _head_body(p0_ref, p1_ref, b2_ref, wm0_ref, bm0_ref, wm1_ref, bm1_ref,
               logits_ref, probs_ref, emb_ref):
    emb = jnp.maximum(p0_ref[...] + p1_ref[...] + b2_ref[...], 0.0)
    emb_ref[...] = emb
    z = jnp.dot(emb, wm0_ref[...], preferred_element_type=jnp.float32) + bm0_ref[...]
    m = jnp.where(z > 0, z, jnp.exp(jnp.minimum(z, 0.0)) - 1.0)
    logits = jnp.dot(m, wm1_ref[...], preferred_element_type=jnp.float32) + bm1_ref[...]
    logits_ref[...] = logits
    lmax = jnp.max(logits, axis=-1, keepdims=True)
    ex = jnp.exp(logits - lmax)
    probs_ref[...] = ex / jnp.sum(ex, axis=-1, keepdims=True)


_head = pl.pallas_call(
    _head_body,
    grid=(N // BLK,),
    in_specs=[
        pl.BlockSpec((BLK, D), lambda i: (i, 0)),
        pl.BlockSpec((BLK, D), lambda i: (N // BLK + i, 0)),
        pl.BlockSpec((1, D), lambda i: (0, 0)),
        pl.BlockSpec((D, MLP_H), lambda i: (0, 0)),
        pl.BlockSpec((1, MLP_H), lambda i: (0, 0)),
        pl.BlockSpec((MLP_H, D_OUT), lambda i: (0, 0)),
        pl.BlockSpec((1, D_OUT), lambda i: (0, 0)),
    ],
    out_specs=[
        pl.BlockSpec((BLK, D_OUT), lambda i: (i, 0)),
        pl.BlockSpec((BLK, D_OUT), lambda i: (i, 0)),
        pl.BlockSpec((BLK, D), lambda i: (i, 0)),
    ],
    out_shape=[
        jax.ShapeDtypeStruct((N, D_OUT), jnp.float32),
        jax.ShapeDtypeStruct((N, D_OUT), jnp.float32),
        jax.ShapeDtypeStruct((N, D), jnp.float32),
    ],
)


def kernel(x, edge_index, W0, b0, W1, b1, W2, b2, Wm0, bm0, Wm1, bm1):
    src = edge_index[0]
    dst = edge_index[1]
    zeros_tile = jnp.zeros((ROWS_LAST, D), jnp.float32)

    h0 = _mm(x, W0)
    a0 = _sc_scatter(h0, src, dst, zeros_tile)
    h1 = _agg_mm(a0, a0, b0.reshape(1, D), W1)
    a1 = _sc_scatter(h1, src, dst, zeros_tile)
    h2 = _agg_mm(a1, a1, b1.reshape(1, D), W2)
    a2 = _sc_scatter(h2, src, dst, zeros_tile)
    logits, probs, emb = _head(
        a2, a2, b2.reshape(1, D), Wm0, bm0.reshape(1, MLP_H),
        Wm1, bm1.reshape(1, D_OUT),
    )
    return (logits, probs, emb)


# speedup vs baseline: 8.0067x; 1.7671x over previous
import functools

import jax
import jax.numpy as jnp
from jax import lax
from jax.experimental import pallas as pl
from jax.experimental.pallas import tpu as pltpu
from jax.experimental.pallas import tpu_sc as plsc

N = 10000
E = 320000
D = 128
MLP_H = 64
D_OUT = 40

NC = 2
NS = 16
NW = NC * NS
E_PER_W = E // NW
CHUNK = 40
NCHUNKS = E_PER_W // CHUNK
ROWS_A = 624
ROWS_LAST = N - (NS - 1) * ROWS_A

_mesh = plsc.VectorSubcoreMesh(
    core_axis_name="c", subcore_axis_name="s", num_cores=NC, num_subcores=NS
)


NBUF = 2
NPHASE = 2
HALF_E = E_PER_W // NPHASE
HALF_CHUNKS = NCHUNKS // NPHASE
HALF_MAIN = (HALF_CHUNKS // NBUF) * NBUF


@functools.partial(
    pl.kernel,
    out_type=jax.ShapeDtypeStruct((NC * N, D), jnp.float32),
    mesh=_mesh,
    scratch_types=[
        pltpu.VMEM((HALF_E,), jnp.int32),
        pltpu.VMEM((NCHUNKS, CHUNK), jnp.int32),
        pltpu.VMEM((NBUF, CHUNK, D), jnp.float32),
        pltpu.VMEM_SHARED((N, D), jnp.float32),
        pltpu.SemaphoreType.DMA,
        pltpu.SemaphoreType.DMA,
    ],
)
def _sc_scatter(h, src, dst3, zeros_tile, out, sidx_half, didx_all, rows,
                acc, sem0, sem1):
    cid = lax.axis_index("c")
    sid = lax.axis_index("s")
    wid = sid * NC + cid
    sems = (sem0, sem1)

    pltpu.sync_copy(dst3.at[wid], didx_all)

    @pl.when(sid < NS - 1)
    def _():
        pltpu.sync_copy(zeros_tile.at[pl.ds(0, ROWS_A)],
                        acc.at[pl.ds(sid * ROWS_A, ROWS_A)])

    @pl.when(sid == NS - 1)
    def _():
        pltpu.sync_copy(zeros_tile, acc.at[pl.ds((NS - 1) * ROWS_A, ROWS_LAST)])

    plsc.subcore_barrier()

    def start_gather(j, b):
        pltpu.async_copy(
            h.at[sidx_half.at[pl.ds(j * CHUNK, CHUNK)]], rows.at[b], sems[b])

    def wait_gather(b):
        pltpu.make_async_copy(h.at[pl.ds(0, CHUNK)], rows.at[b], sems[b]).wait()

    for p in range(NPHASE):
        pltpu.sync_copy(src.at[pl.ds(wid * E_PER_W + p * HALF_E, HALF_E)],
                        sidx_half)
        cbase = p * HALF_CHUNKS

        def scatter(j, b):
            pltpu.sync_copy(rows.at[b], acc.at[didx_all.at[cbase + j]], add=True)

        for b in range(NBUF):
            start_gather(b, b)

        def body(i, carry):
            j0 = i * NBUF
            for b in range(NBUF):
                j = j0 + b
                wait_gather(b)
                scatter(j, b)

                @pl.when(j + NBUF < HALF_CHUNKS)
                def _():
                    start_gather(j + NBUF, b)
            return carry

        lax.fori_loop(0, HALF_MAIN // NBUF, body, 0)
        for j in range(HALF_MAIN, HALF_CHUNKS):
            b = j - HALF_MAIN
            wait_gather(b)
            scatter(j, b)
    plsc.subcore_barrier()

    @pl.when(sid < NS - 1)
    def _():
        pltpu.sync_copy(
            acc.at[pl.ds(sid * ROWS_A, ROWS_A)],
            out.at[pl.ds(cid * N + sid * ROWS_A, ROWS_A)],
        )

    @pl.when(sid == NS - 1)
    def _():
        pltpu.sync_copy(
            acc.at[pl.ds((NS - 1) * ROWS_A, ROWS_LAST)],
            out.at[pl.ds(cid * N + (NS - 1) * ROWS_A, ROWS_LAST)],
        )


BLK = 2000


def _mm_body(x_ref, w_ref, o_ref):
    o_ref[...] = jnp.dot(x_ref[...], w_ref[...], preferred_element_type=jnp.float32)


_mm = pl.pallas_call(
    _mm_body,
    grid=(N // BLK,),
    in_specs=[
        pl.BlockSpec((BLK, D), lambda i: (i, 0)),
        pl.BlockSpec((D, D), lambda i: (0, 0)),
    ],
    out_specs=pl.BlockSpec((BLK, D), lambda i: (i, 0)),
    out_shape=jax.ShapeDtypeStruct((N, D), jnp.float32),
)


def _agg_mm_body(p0_ref, p1_ref, b_ref, w_ref, o_ref):
    h = jnp.maximum(p0_ref[...] + p1_ref[...] + b_ref[...], 0.0)
    o_ref[...] = jnp.dot(h, w_ref[...], preferred_element_type=jnp.float32)


_agg_mm = pl.pallas_call(
    _agg_mm_body,
    grid=(N // BLK,),
    in_specs=[
        pl.BlockSpec((BLK, D), lambda i: (i, 0)),
        pl.BlockSpec((BLK, D), lambda i: (N // BLK + i, 0)),
        pl.BlockSpec((1, D), lambda i: (0, 0)),
        pl.BlockSpec((D, D), lambda i: (0, 0)),
    ],
    out_specs=pl.BlockSpec((BLK, D), lambda i: (i, 0)),
    out_shape=jax.ShapeDtypeStruct((N, D), jnp.float32),
)


def _head_body(p0_ref, p1_ref, b2_ref, wm0_ref, bm0_ref, wm1_ref, bm1_ref,
               logits_ref, probs_ref, emb_ref):
    emb = jnp.maximum(p0_ref[...] + p1_ref[...] + b2_ref[...], 0.0)
    emb_ref[...] = emb
    z = jnp.dot(emb, wm0_ref[...], preferred_element_type=jnp.float32) + bm0_ref[...]
    m = jnp.where(z > 0, z, jnp.exp(jnp.minimum(z, 0.0)) - 1.0)
    logits = jnp.dot(m, wm1_ref[...], preferred_element_type=jnp.float32) + bm1_ref[...]
    logits_ref[...] = logits
    lmax = jnp.max(logits, axis=-1, keepdims=True)
    ex = jnp.exp(logits - lmax)
    probs_ref[...] = ex / jnp.sum(ex, axis=-1, keepdims=True)


_head = pl.pallas_call(
    _head_body,
    grid=(N // BLK,),
    in_specs=[
        pl.BlockSpec((BLK, D), lambda i: (i, 0)),
        pl.BlockSpec((BLK, D), lambda i: (N // BLK + i, 0)),
        pl.BlockSpec((1, D), lambda i: (0, 0)),
        pl.BlockSpec((D, MLP_H), lambda i: (0, 0)),
        pl.BlockSpec((1, MLP_H), lambda i: (0, 0)),
        pl.BlockSpec((MLP_H, D_OUT), lambda i: (0, 0)),
        pl.BlockSpec((1, D_OUT), lambda i: (0, 0)),
    ],
    out_specs=[
        pl.BlockSpec((BLK, D_OUT), lambda i: (i, 0)),
        pl.BlockSpec((BLK, D_OUT), lambda i: (i, 0)),
        pl.BlockSpec((BLK, D), lambda i: (i, 0)),
    ],
    out_shape=[
        jax.ShapeDtypeStruct((N, D_OUT), jnp.float32),
        jax.ShapeDtypeStruct((N, D_OUT), jnp.float32),
        jax.ShapeDtypeStruct((N, D), jnp.float32),
    ],
)


def kernel(x, edge_index, W0, b0, W1, b1, W2, b2, Wm0, bm0, Wm1, bm1):
    src = edge_index[0]
    dst3 = edge_index[1].reshape(NW, NCHUNKS, CHUNK)
    zeros_tile = jnp.zeros((ROWS_LAST, D), jnp.float32)

    h0 = _mm(x, W0)
    a0 = _sc_scatter(h0, src, dst3, zeros_tile)
    h1 = _agg_mm(a0, a0, b0.reshape(1, D), W1)
    a1 = _sc_scatter(h1, src, dst3, zeros_tile)
    h2 = _agg_mm(a1, a1, b1.reshape(1, D), W2)
    a2 = _sc_scatter(h2, src, dst3, zeros_tile)
    logits, probs, emb = _head(
        a2, a2, b2.reshape(1, D), Wm0, bm0.reshape(1, MLP_H),
        Wm1, bm1.reshape(1, D_OUT),
    )
    return (logits, probs, emb)


# speedup vs baseline: 10.9380x; 1.3661x over previous
import functools

import jax
import jax.numpy as jnp
from jax import lax
from jax.experimental import pallas as pl
from jax.experimental.pallas import tpu as pltpu
from jax.experimental.pallas import tpu_sc as plsc

N = 10000
E = 320000
D = 128
MLP_H = 64
D_OUT = 40

NC = 2
NS = 16
NW = NC * NS
E_PER_W = E // NW
CHUNK = 40
NCHUNKS = E_PER_W // CHUNK
ROWS_A = 624
ROWS_LAST = N - (NS - 1) * ROWS_A

_mesh = plsc.VectorSubcoreMesh(
    core_axis_name="c", subcore_axis_name="s", num_cores=NC, num_subcores=NS
)


NBUF = 4
NPHASE = 5
HALF_E = E_PER_W // NPHASE
HALF_CHUNKS = NCHUNKS // NPHASE
HALF_MAIN = (HALF_CHUNKS // NBUF) * NBUF


@functools.partial(
    pl.kernel,
    out_type=jax.ShapeDtypeStruct((NC * N, D), jnp.float32),
    mesh=_mesh,
    scratch_types=[
        pltpu.VMEM((HALF_E,), jnp.int32),
        pltpu.VMEM((HALF_CHUNKS, CHUNK), jnp.int32),
        pltpu.VMEM((NBUF, CHUNK, D), jnp.float32),
        pltpu.VMEM_SHARED((N, D), jnp.float32),
        pltpu.SemaphoreType.DMA,
        pltpu.SemaphoreType.DMA,
        pltpu.SemaphoreType.DMA,
        pltpu.SemaphoreType.DMA,
    ],
)
def _sc_scatter(h, src, dst4, zeros_tile, out, sidx_half, didx_half, rows,
                acc, sem0, sem1, sem2, sem3):
    cid = lax.axis_index("c")
    sid = lax.axis_index("s")
    wid = sid * NC + cid
    sems = (sem0, sem1, sem2, sem3)

    @pl.when(sid < NS - 1)
    def _():
        pltpu.sync_copy(zeros_tile.at[pl.ds(0, ROWS_A)],
                        acc.at[pl.ds(sid * ROWS_A, ROWS_A)])

    @pl.when(sid == NS - 1)
    def _():
        pltpu.sync_copy(zeros_tile, acc.at[pl.ds((NS - 1) * ROWS_A, ROWS_LAST)])

    plsc.subcore_barrier()

    def start_gather(j, b):
        pltpu.async_copy(
            h.at[sidx_half.at[pl.ds(j * CHUNK, CHUNK)]], rows.at[b], sems[b])

    def wait_gather(b):
        pltpu.make_async_copy(h.at[pl.ds(0, CHUNK)], rows.at[b], sems[b]).wait()

    for p in range(NPHASE):
        pltpu.sync_copy(src.at[pl.ds(wid * E_PER_W + p * HALF_E, HALF_E)],
                        sidx_half)
        pltpu.sync_copy(dst4.at[wid, p], didx_half)

        def scatter(j, b):
            pltpu.sync_copy(rows.at[b], acc.at[didx_half.at[j]], add=True)

        for b in range(NBUF):
            start_gather(b, b)

        def body(i, carry):
            j0 = i * NBUF
            for b in range(NBUF):
                j = j0 + b
                wait_gather(b)
                scatter(j, b)

                @pl.when(j + NBUF < HALF_CHUNKS)
                def _():
                    start_gather(j + NBUF, b)
            return carry

        lax.fori_loop(0, HALF_MAIN // NBUF, body, 0)
        for j in range(HALF_MAIN, HALF_CHUNKS):
            b = j - HALF_MAIN
            wait_gather(b)
            scatter(j, b)
    plsc.subcore_barrier()

    @pl.when(sid < NS - 1)
    def _():
        pltpu.sync_copy(
            acc.at[pl.ds(sid * ROWS_A, ROWS_A)],
            out.at[pl.ds(cid * N + sid * ROWS_A, ROWS_A)],
        )

    @pl.when(sid == NS - 1)
    def _():
        pltpu.sync_copy(
            acc.at[pl.ds((NS - 1) * ROWS_A, ROWS_LAST)],
            out.at[pl.ds(cid * N + (NS - 1) * ROWS_A, ROWS_LAST)],
        )


BLK = 2000


def _mm_body(x_ref, w_ref, o_ref):
    o_ref[...] = jnp.dot(x_ref[...], w_ref[...], preferred_element_type=jnp.float32)


_mm = pl.pallas_call(
    _mm_body,
    grid=(N // BLK,),
    in_specs=[
        pl.BlockSpec((BLK, D), lambda i: (i, 0)),
        pl.BlockSpec((D, D), lambda i: (0, 0)),
    ],
    out_specs=pl.BlockSpec((BLK, D), lambda i: (i, 0)),
    out_shape=jax.ShapeDtypeStruct((N, D), jnp.float32),
)


def _agg_mm_body(p0_ref, p1_ref, b_ref, w_ref, o_ref):
    h = jnp.maximum(p0_ref[...] + p1_ref[...] + b_ref[...], 0.0)
    o_ref[...] = jnp.dot(h, w_ref[...], preferred_element_type=jnp.float32)


_agg_mm = pl.pallas_call(
    _agg_mm_body,
    grid=(N // BLK,),
    in_specs=[
        pl.BlockSpec((BLK, D), lambda i: (i, 0)),
        pl.BlockSpec((BLK, D), lambda i: (N // BLK + i, 0)),
        pl.BlockSpec((1, D), lambda i: (0, 0)),
        pl.BlockSpec((D, D), lambda i: (0, 0)),
    ],
    out_specs=pl.BlockSpec((BLK, D), lambda i: (i, 0)),
    out_shape=jax.ShapeDtypeStruct((N, D), jnp.float32),
)


def _head_body(p0_ref, p1_ref, b2_ref, wm0_ref, bm0_ref, wm1_ref, bm1_ref,
               logits_ref, probs_ref, emb_ref):
    emb = jnp.maximum(p0_ref[...] + p1_ref[...] + b2_ref[...], 0.0)
    emb_ref[...] = emb
    z = jnp.dot(emb, wm0_ref[...], preferred_element_type=jnp.float32) + bm0_ref[...]
    m = jnp.where(z > 0, z, jnp.exp(jnp.minimum(z, 0.0)) - 1.0)
    logits = jnp.dot(m, wm1_ref[...], preferred_element_type=jnp.float32) + bm1_ref[...]
    logits_ref[...] = logits
    lmax = jnp.max(logits, axis=-1, keepdims=True)
    ex = jnp.exp(logits - lmax)
    probs_ref[...] = ex / jnp.sum(ex, axis=-1, keepdims=True)


_head = pl.pallas_call(
    _head_body,
    grid=(N // BLK,),
    in_specs=[
        pl.BlockSpec((BLK, D), lambda i: (i, 0)),
        pl.BlockSpec((BLK, D), lambda i: (N // BLK + i, 0)),
        pl.BlockSpec((1, D), lambda i: (0, 0)),
        pl.BlockSpec((D, MLP_H), lambda i: (0, 0)),
        pl.BlockSpec((1, MLP_H), lambda i: (0, 0)),
        pl.BlockSpec((MLP_H, D_OUT), lambda i: (0, 0)),
        pl.BlockSpec((1, D_OUT), lambda i: (0, 0)),
    ],
    out_specs=[
        pl.BlockSpec((BLK, D_OUT), lambda i: (i, 0)),
        pl.BlockSpec((BLK, D_OUT), lambda i: (i, 0)),
        pl.BlockSpec((BLK, D), lambda i: (i, 0)),
    ],
    out_shape=[
        jax.ShapeDtypeStruct((N, D_OUT), jnp.float32),
        jax.ShapeDtypeStruct((N, D_OUT), jnp.float32),
        jax.ShapeDtypeStruct((N, D), jnp.float32),
    ],
)


def kernel(x, edge_index, W0, b0, W1, b1, W2, b2, Wm0, bm0, Wm1, bm1):
    src = edge_index[0]
    dst4 = edge_index[1].reshape(NW, NPHASE, HALF_CHUNKS, CHUNK)
    zeros_tile = jnp.zeros((ROWS_LAST, D), jnp.float32)

    h0 = _mm(x, W0)
    a0 = _sc_scatter(h0, src, dst4, zeros_tile)
    h1 = _agg_mm(a0, a0, b0.reshape(1, D), W1)
    a1 = _sc_scatter(h1, src, dst4, zeros_tile)
    h2 = _agg_mm(a1, a1, b1.reshape(1, D), W2)
    a2 = _sc_scatter(h2, src, dst4, zeros_tile)
    logits, probs, emb = _head(
        a2, a2, b2.reshape(1, D), Wm0, bm0.reshape(1, MLP_H),
        Wm1, bm1.reshape(1, D_OUT),
    )
    return (logits, probs, emb)
